# baseline (speedup 1.0000x reference)
import functools

import jax
import jax.numpy as jnp
from jax import lax
from jax.experimental import pallas as pl
from jax.experimental.pallas import tpu as pltpu
from jax.experimental.pallas import tpu_sc as plsc

VOCAB = 205
D = 16
CHUNK = 1024
NB = 4
NC, NS = 2, 16
NW = NC * NS


def _build(B):
    assert B % (NW * CHUNK) == 0
    per_w = B // NW
    nchunks = per_w // CHUNK
    assert nchunks % NB == 0
    nrounds = nchunks // NB

    mesh = plsc.VectorSubcoreMesh(core_axis_name="c", subcore_axis_name="s")

    @functools.partial(
        pl.kernel,
        out_type=jax.ShapeDtypeStruct((B, D), jnp.float32),
        mesh=mesh,
        scratch_types=(
            [pltpu.VMEM((NB, CHUNK), jnp.int32),
             pltpu.VMEM((NB, CHUNK, D), jnp.float32)]
            + [pltpu.SemaphoreType.DMA] * NB
            + [pltpu.SemaphoreType.DMA] * NB
            + [pltpu.SemaphoreType.DMA] * NB
        ),
        compiler_params=pltpu.CompilerParams(use_tc_tiling_on_sc=False),
    )
    def k(x_hbm, table_hbm, out_hbm, idx_v, rows_v, *sems):
        sem_i = sems[:NB]
        sem_g = sems[NB:2 * NB]
        sem_w = sems[2 * NB:]
        wid = lax.axis_index("s") * NC + lax.axis_index("c")
        row0 = wid * per_w

        def idx_load(g, b):
            return pltpu.make_async_copy(
                x_hbm.at[pl.ds(row0 + g * CHUNK, CHUNK)], idx_v.at[b],
                sem_i[b])

        def gather(g, b):
            return pltpu.make_async_copy(
                table_hbm.at[idx_v.at[b]], rows_v.at[b], sem_g[b])

        def write(g, b):
            return pltpu.make_async_copy(
                rows_v.at[b], out_hbm.at[pl.ds(row0 + g * CHUNK, CHUNK)],
                sem_w[b])

        for b in range(NB):
            idx_load(b, b).start()

        def round_body(r, _):
            for b in range(NB):
                g = r * NB + b
                idx_load(g, b).wait()
                gather(g, b).start()
            for b in range(NB):
                g = r * NB + b
                gather(g, b).wait()
                write(g, b).start()
            for b in range(NB):
                g = r * NB + b
                write(g, b).wait()
                idx_load(g + NB, b).start()
            return 0

        lax.fori_loop(0, nrounds - 1, round_body, 0)

        r = nrounds - 1
        for b in range(NB):
            g = r * NB + b
            idx_load(g, b).wait()
            gather(g, b).start()
        for b in range(NB):
            g = r * NB + b
            gather(g, b).wait()
            write(g, b).start()
        for b in range(NB):
            g = r * NB + b
            write(g, b).wait()

    return k


def kernel(x, embed_site):
    B = x.size
    return _build(B)(x.reshape(B).astype(jnp.int32), embed_site)


# speedup vs baseline: 1.3773x; 1.3773x over previous
import functools

import jax
import jax.numpy as jnp
from jax import lax
from jax.experimental import pallas as pl
from jax.experimental.pallas import tpu as pltpu
from jax.experimental.pallas import tpu_sc as plsc

VOCAB = 205
D = 16
CHUNK = 1024
NB = 4
NC, NS = 2, 16
NW = NC * NS


def _build(B):
    assert B % (NW * CHUNK) == 0
    per_w = B // NW
    nchunks = per_w // CHUNK
    assert nchunks % NB == 0
    nrounds = nchunks // NB

    mesh = plsc.VectorSubcoreMesh(core_axis_name="c", subcore_axis_name="s")

    @functools.partial(
        pl.kernel,
        out_type=jax.ShapeDtypeStruct((B, D), jnp.float32),
        mesh=mesh,
        scratch_types=(
            [pltpu.VMEM((NB, CHUNK), jnp.int32),
             pltpu.VMEM((NB, CHUNK, D), jnp.float32),
             pltpu.VMEM_SHARED((VOCAB, D), jnp.float32)]
            + [pltpu.SemaphoreType.DMA] * NB
            + [pltpu.SemaphoreType.DMA] * NB
            + [pltpu.SemaphoreType.DMA] * NB
        ),
        compiler_params=pltpu.CompilerParams(use_tc_tiling_on_sc=False),
    )
    def k(x_hbm, table_hbm, out_hbm, idx_v, rows_v, tbl_sh, *sems):
        sem_i = sems[:NB]
        sem_g = sems[NB:2 * NB]
        sem_w = sems[2 * NB:]
        sid = lax.axis_index("s")
        wid = sid * NC + lax.axis_index("c")
        row0 = wid * per_w

        @pl.when(sid == 0)
        def _():
            pltpu.sync_copy(table_hbm, tbl_sh)

        plsc.subcore_barrier()

        def idx_load(g, b):
            return pltpu.make_async_copy(
                x_hbm.at[pl.ds(row0 + g * CHUNK, CHUNK)], idx_v.at[b],
                sem_i[b])

        def gather(g, b):
            return pltpu.make_async_copy(
                tbl_sh.at[idx_v.at[b]], rows_v.at[b], sem_g[b])

        def write(g, b):
            return pltpu.make_async_copy(
                rows_v.at[b], out_hbm.at[pl.ds(row0 + g * CHUNK, CHUNK)],
                sem_w[b])

        for b in range(NB):
            idx_load(b, b).start()

        def round_body(r, _):
            for b in range(NB):
                g = r * NB + b
                idx_load(g, b).wait()
                gather(g, b).start()
            for b in range(NB):
                g = r * NB + b
                gather(g, b).wait()
                write(g, b).start()
            for b in range(NB):
                g = r * NB + b
                write(g, b).wait()
                idx_load(g + NB, b).start()
            return 0

        lax.fori_loop(0, nrounds - 1, round_body, 0)

        r = nrounds - 1
        for b in range(NB):
            g = r * NB + b
            idx_load(g, b).wait()
            gather(g, b).start()
        for b in range(NB):
            g = r * NB + b
            gather(g, b).wait()
            write(g, b).start()
        for b in range(NB):
            g = r * NB + b
            write(g, b).wait()

    return k


def kernel(x, embed_site):
    B = x.size
    return _build(B)(x.reshape(B).astype(jnp.int32), embed_site)
